# Initial kernel scaffold; baseline (speedup 1.0000x reference)
import jax
import jax.numpy as jnp
from jax.experimental import pallas as pl


def kernel(input, offsets, W):
    raise NotImplementedError("write your pallas kernel here")


# speedup vs baseline: 938.6231x; 938.6231x over previous
import functools

import jax
import jax.numpy as jnp
from jax.experimental import pallas as pl
from jax.experimental.pallas import tpu as pltpu
from jax.experimental.pallas import tpu_sc as plsc


def _qbuild_body(x_ref, q_ref):
    xr = x_ref[...]
    m, H, W, C = xr.shape
    xj = jnp.concatenate([xr[:, :, 1:, :], xr[:, :, W - 1:W, :]], axis=2)
    xi = jnp.concatenate([xr[:, 1:, :, :], xr[:, H - 1:H, :, :]], axis=1)
    xij = jnp.concatenate([xi[:, :, 1:, :], xi[:, :, W - 1:W, :]], axis=2)
    q = jnp.concatenate([xr, xj, xi, xij], axis=3)
    q_ref[...] = q.reshape(m * H * W, 4 * C)


def _sc_gather(q, idx, gwin):
    S = idx.shape[1]
    D = q.shape[1]
    mesh = plsc.VectorSubcoreMesh(core_axis_name="c", subcore_axis_name="s")

    @functools.partial(
        pl.kernel,
        out_type=jax.ShapeDtypeStruct((S, D), q.dtype),
        mesh=mesh,
    )
    def gather_kernel(q_hbm, i_hbm, o_hbm):
        def body(i_vmem, o_vmem):
            pltpu.sync_copy(q_hbm.at[i_vmem.at[0]], o_vmem)

        pltpu.emit_pipeline(
            body,
            grid=(S // gwin,),
            in_specs=[pl.BlockSpec((1, gwin), lambda i: (0, i))],
            out_specs=[pl.BlockSpec((gwin, D), lambda i: (i, 0))],
            core_axis_name=("c", "s"),
            dimension_semantics=(pltpu.PARALLEL,),
        )(i_hbm, o_hbm)

    return gather_kernel(q, idx)


def _combine_body(g_ref, fi_ref, fj_ref, w_ref, o_ref, *, P, C, N9, F):
    acc = jnp.zeros((P, F), jnp.float32)
    for n in range(N9):
        gn = g_ref[pl.ds(n * P, P), :]
        v_lt = gn[:, 0:C]
        v_rt = gn[:, C:2 * C]
        v_lb = gn[:, 2 * C:3 * C]
        v_rb = gn[:, 3 * C:4 * C]
        fjn = fj_ref[:, n:n + 1]
        fin = fi_ref[:, n:n + 1]
        v_t = v_lt + (v_rt - v_lt) * fjn
        v_b = v_lb + (v_rb - v_lb) * fjn
        io = v_t + (v_b - v_t) * fin
        acc = acc + jnp.dot(io, w_ref[n], preferred_element_type=jnp.float32)
    o_ref[...] = acc


def kernel(input, offsets, W):
    x = input
    m, H, Wd, C = x.shape
    N9 = offsets.shape[3] // 2
    F = W.shape[2]
    P = m * H * Wd
    S = N9 * P

    off = offsets.reshape(P, N9, 2)
    pos = jnp.arange(P, dtype=jnp.int32)
    ii = (pos % (H * Wd)) // Wd
    jj = pos % Wd
    mb = pos // (H * Wd)
    ci = jnp.clip(ii[:, None].astype(jnp.float32) + off[:, :, 0], 0.0, float(H - 1))
    cj = jnp.clip(jj[:, None].astype(jnp.float32) + off[:, :, 1], 0.0, float(Wd - 1))
    lt_i = jnp.floor(ci)
    lt_j = jnp.floor(cj)
    fi = ci - lt_i
    fj = cj - lt_j
    idxmat = mb[:, None] * (H * Wd) + lt_i.astype(jnp.int32) * Wd \
        + lt_j.astype(jnp.int32)
    gwin = 128
    s_pad = ((S + gwin - 1) // gwin) * gwin
    idx_nm = jnp.pad(idxmat.T.reshape(1, S), ((0, 0), (0, s_pad - S)))

    q = pl.pallas_call(
        _qbuild_body,
        out_shape=jax.ShapeDtypeStruct((P, 4 * C), jnp.float32),
    )(x)

    g = _sc_gather(q, idx_nm, gwin=gwin)

    w2 = W.transpose(1, 0, 2)
    out = pl.pallas_call(
        functools.partial(_combine_body, P=P, C=C, N9=N9, F=F),
        out_shape=jax.ShapeDtypeStruct((P, F), jnp.float32),
    )(g, fi, fj, w2)

    return out.reshape(m, H, Wd, F)
